# baseline (speedup 1.0000x reference)
import functools

import jax
import jax.numpy as jnp
from jax import lax
from jax.experimental import pallas as pl
from jax.experimental.pallas import tpu as pltpu
from jax.experimental.pallas import tpu_sc as plsc

L = 16
G = 16
B = 4
N = 50000
D = 128
DL = D // L
SEGS = B * G
TOTAL = B * N
NC, NS = 2, 16
NW = NC * NS
RANGE = TOTAL // NW
CHUNK = 416
NWIN = 16
GRP = 16


def _mesh():
    return plsc.VectorSubcoreMesh(core_axis_name="c", subcore_axis_name="s")


@functools.partial(
    pl.kernel,
    out_type=(
        jax.ShapeDtypeStruct((NW, SEGS, L), jnp.float32),
        jax.ShapeDtypeStruct((NW, SEGS, L), jnp.float32),
    ),
    mesh=_mesh(),
    scratch_types=[
        pltpu.VMEM((CHUNK, D), jnp.float32),
        pltpu.VMEM((CHUNK, D), jnp.float32),
        pltpu.VMEM((CHUNK + L,), jnp.int32),
        pltpu.VMEM((CHUNK + L,), jnp.int32),
        pltpu.VMEM((D,), jnp.float32),
        pltpu.VMEM((SEGS, L), jnp.float32),
        pltpu.VMEM((SEGS, L), jnp.float32),
        pltpu.SemaphoreType.DMA,
        pltpu.SemaphoreType.DMA,
        pltpu.SemaphoreType.DMA,
        pltpu.SemaphoreType.DMA,
    ],
)
def _partials(nodes, gidx, wt, part_out, cnt_out, nbuf0, nbuf1, ibuf0, ibuf1,
              wbuf, acc, cnt, *sems):
    nbufs, ibufs = (nbuf0, nbuf1), (ibuf0, ibuf1)
    w = lax.axis_index("s") * NC + lax.axis_index("c")
    start = w * RANGE
    end = start + RANGE
    a0 = (start // 8) * 8

    pltpu.sync_copy(wt, wbuf)
    zrow = jnp.zeros((L,), jnp.float32)
    ones = jnp.ones((L,), jnp.float32)
    for g in range(SEGS):
        acc[g] = zrow
        cnt[g] = zrow
    wv = [wbuf[pl.ds(d * L, L)] for d in range(DL)]

    def win_base(k):
        wk = a0 + k * CHUNK
        return wk, jnp.minimum(wk, TOTAL - CHUNK)

    def dma_start(k, par):
        _, wkc = win_base(k)
        pltpu.make_async_copy(nodes.at[pl.ds(wkc, CHUNK)], nbufs[par],
                              sems[2 * par]).start()
        pltpu.make_async_copy(gidx.at[pl.ds(wkc, CHUNK)],
                              ibufs[par].at[pl.ds(0, CHUNK)],
                              sems[2 * par + 1]).start()

    def dma_wait(par):
        pltpu.make_async_copy(nodes.at[pl.ds(0, CHUNK)], nbufs[par],
                              sems[2 * par]).wait()
        pltpu.make_async_copy(gidx.at[pl.ds(0, CHUNK)],
                              ibufs[par].at[pl.ds(0, CHUNK)],
                              sems[2 * par + 1]).wait()

    def process(k, par):
        wk, wkc = win_base(k)
        lo = jnp.maximum(start, wk) - wkc
        hi = jnp.minimum(end, wk + CHUNK) - wkc

        def node_dot(n):
            prods = [nbufs[par][n, pl.ds(d * L, L)] * wv[d] for d in range(DL)]
            while len(prods) > 1:
                prods = [prods[i] + prods[i + 1]
                         for i in range(0, len(prods) - 1, 2)] + prods[len(prods) & ~1:]
            return prods[0]

        ngrp = jnp.maximum(hi - lo, 0) // GRP

        @plsc.parallel_loop(0, ngrp)
        def grp_body(gi):
            base = lo + gi * GRP
            gv = ibufs[par][pl.ds(base, L)]
            ts = [node_dot(base + j) for j in range(GRP)]
            for j in range(GRP):
                plsc.addupdate(acc.at[gv[j]], ts[j])
                plsc.addupdate(cnt.at[gv[j]], ones)

        def tail_body(n, _):
            g = ibufs[par][pl.ds(n, L)][0]
            plsc.addupdate(acc.at[g], node_dot(n))
            plsc.addupdate(cnt.at[g], ones)
            return 0

        lax.fori_loop(lo + ngrp * GRP, hi, tail_body, 0)

    dma_start(0, 0)

    def pair_body(p, _):
        for par in range(2):
            k = 2 * p + par

            @pl.when(k + 1 < NWIN)
            def _():
                dma_start(k + 1, 1 - par)

            dma_wait(par)
            process(k, par)
        return 0

    lax.fori_loop(0, NWIN // 2, pair_body, 0)
    pltpu.sync_copy(acc, part_out.at[w])
    pltpu.sync_copy(cnt, cnt_out.at[w])


@functools.partial(
    pl.kernel,
    out_type=jax.ShapeDtypeStruct((SEGS, L), jnp.float32),
    mesh=_mesh(),
    scratch_types=[
        pltpu.VMEM((NW, 2, L), jnp.float32),
        pltpu.VMEM((NW, 2, L), jnp.float32),
        pltpu.VMEM((L,), jnp.float32),
        pltpu.VMEM((2 * L,), jnp.float32),
        pltpu.VMEM((2, L), jnp.float32),
    ],
)
def _finish(part, cnts, bt, out, pbuf, cbuf, bbuf, red, obuf):
    w = lax.axis_index("s") * NC + lax.axis_index("c")
    seg0 = w * 2
    pltpu.sync_copy(part.at[:, pl.ds(seg0, 2), :], pbuf)
    pltpu.sync_copy(cnts.at[:, pl.ds(seg0, 2), :], cbuf)
    pltpu.sync_copy(bt, bbuf)

    zrow = jnp.zeros((L,), jnp.float32)

    def pbody(p, carry):
        s0, c0, s1, c1 = carry
        return (s0 + pbuf[p, 0], c0 + cbuf[p, 0],
                s1 + pbuf[p, 1], c1 + cbuf[p, 1])

    s0, c0, s1, c1 = lax.fori_loop(0, NW, pbody, (zrow, zrow, zrow, zrow))

    bias = bbuf[...]
    red[pl.ds(L, L)] = zrow
    for i, (s, c) in enumerate(((s0, c0), (s1, c1))):
        red[pl.ds(0, L)] = s
        for sh in (8, 4, 2, 1):
            red[pl.ds(0, L)] = red[pl.ds(0, L)] + red[pl.ds(sh, L)]
        tot = red[pl.ds(0, L)]
        obuf[i] = tot / jnp.maximum(c, 1.0) + bias

    pltpu.sync_copy(obuf, out.at[pl.ds(seg0, 2)])


def kernel(nodes, edges, receivers, senders, global_latent, node_graph_idx,
           edge_graph_idx, W, b):
    flat_nodes = nodes.reshape(B * N, D)
    flat_idx = (node_graph_idx
                + (jnp.arange(B, dtype=jnp.int32) * G)[:, None]).reshape(-1)
    wt = W.reshape(D)
    bt = jnp.broadcast_to(b.astype(jnp.float32), (L,))
    part, cnt = _partials(flat_nodes, flat_idx, wt)
    res = _finish(part, cnt, bt)
    return res.reshape(B, G, L)[..., :1]


# speedup vs baseline: 1.4641x; 1.4641x over previous
import functools

import jax
import jax.numpy as jnp
from jax import lax
from jax.experimental import pallas as pl
from jax.experimental.pallas import tpu as pltpu
from jax.experimental.pallas import tpu_sc as plsc

L = 16
G = 16
B = 4
N = 50000
D = 128
DL = D // L
SEGS = B * G
TOTAL = B * N
NC, NS = 2, 16
NW = NC * NS
S_SC = 64000
RANGE = S_SC // NW
CHUNK = 256
NWIN = -(-(RANGE + 7) // CHUNK) + (-(-(RANGE + 7) // CHUNK) & 1)
GRP = 16
TC_BLK = 2000
TC_OFF = S_SC // TC_BLK
TC_NBLK = (TOTAL - S_SC) // TC_BLK


def _mesh():
    return plsc.VectorSubcoreMesh(core_axis_name="c", subcore_axis_name="s")


@functools.partial(
    pl.kernel,
    out_type=(
        jax.ShapeDtypeStruct((NW, SEGS, L), jnp.float32),
        jax.ShapeDtypeStruct((NW, SEGS, L), jnp.float32),
    ),
    mesh=_mesh(),
    scratch_types=[
        pltpu.VMEM((CHUNK, D), jnp.float32),
        pltpu.VMEM((CHUNK, D), jnp.float32),
        pltpu.VMEM((CHUNK + L,), jnp.int32),
        pltpu.VMEM((CHUNK + L,), jnp.int32),
        pltpu.VMEM((D,), jnp.float32),
        pltpu.VMEM((SEGS, L), jnp.float32),
        pltpu.VMEM((SEGS, L), jnp.float32),
        pltpu.SemaphoreType.DMA,
        pltpu.SemaphoreType.DMA,
        pltpu.SemaphoreType.DMA,
        pltpu.SemaphoreType.DMA,
    ],
)
def _partials(nodes, gidx, wt, part_out, cnt_out, nbuf0, nbuf1, ibuf0, ibuf1,
              wbuf, acc, cnt, *sems):
    nbufs, ibufs = (nbuf0, nbuf1), (ibuf0, ibuf1)
    w = lax.axis_index("s") * NC + lax.axis_index("c")
    start = w * RANGE
    end = start + RANGE
    a0 = (start // 8) * 8

    pltpu.sync_copy(wt, wbuf)
    zrow = jnp.zeros((L,), jnp.float32)
    ones = jnp.ones((L,), jnp.float32)
    for g in range(SEGS):
        acc[g] = zrow
        cnt[g] = zrow
    wv = [wbuf[pl.ds(d * L, L)] for d in range(DL)]

    def win_base(k):
        wk = a0 + k * CHUNK
        return wk, jnp.minimum(wk, TOTAL - CHUNK)

    def dma_start(k, par):
        _, wkc = win_base(k)
        pltpu.make_async_copy(nodes.at[pl.ds(wkc, CHUNK)], nbufs[par],
                              sems[2 * par]).start()
        pltpu.make_async_copy(gidx.at[pl.ds(wkc, CHUNK)],
                              ibufs[par].at[pl.ds(0, CHUNK)],
                              sems[2 * par + 1]).start()

    def dma_wait(par):
        pltpu.make_async_copy(nodes.at[pl.ds(0, CHUNK)], nbufs[par],
                              sems[2 * par]).wait()
        pltpu.make_async_copy(gidx.at[pl.ds(0, CHUNK)],
                              ibufs[par].at[pl.ds(0, CHUNK)],
                              sems[2 * par + 1]).wait()

    def process(k, par):
        wk, wkc = win_base(k)
        lo = jnp.maximum(start, wk) - wkc
        hi = jnp.minimum(end, wk + CHUNK) - wkc

        def node_dot(n):
            prods = [nbufs[par][n, pl.ds(d * L, L)] * wv[d] for d in range(DL)]
            while len(prods) > 1:
                prods = [prods[i] + prods[i + 1]
                         for i in range(0, len(prods) - 1, 2)] + prods[len(prods) & ~1:]
            return prods[0]

        @plsc.parallel_loop(lo, hi, unroll=GRP)
        def node_body(n):
            g = ibufs[par][pl.ds(n, L)][0]
            plsc.addupdate(acc.at[g], node_dot(n))
            plsc.addupdate(cnt.at[g], ones)

    dma_start(0, 0)

    def pair_body(p, _):
        for par in range(2):
            k = 2 * p + par

            @pl.when(k + 1 < NWIN)
            def _():
                dma_start(k + 1, 1 - par)

            dma_wait(par)
            process(k, par)
        return 0

    lax.fori_loop(0, NWIN // 2, pair_body, 0)
    pltpu.sync_copy(acc, part_out.at[w])
    pltpu.sync_copy(cnt, cnt_out.at[w])


def _tc_body(idx_ref, nodes_ref, we1_ref, outy_ref, outc_ref):
    i = pl.program_id(0)

    @pl.when(i == 0)
    def _():
        outy_ref[...] = jnp.zeros_like(outy_ref)
        outc_ref[...] = jnp.zeros_like(outc_ref)

    x = nodes_ref[...]
    ids = idx_ref[0, 0, :]
    iota = lax.broadcasted_iota(jnp.int32, (SEGS, TC_BLK), 0)
    oh = (iota == ids[None, :]).astype(jnp.float32)
    y16 = jnp.dot(x, we1_ref[...], preferred_element_type=jnp.float32)
    outy_ref[...] += jnp.dot(oh, y16, preferred_element_type=jnp.float32)
    lane0 = (lax.broadcasted_iota(jnp.int32, (SEGS, L), 1) == 0)
    cnts = jnp.sum(oh, axis=1, keepdims=True)
    outc_ref[...] += jnp.where(lane0, cnts, 0.0)


_tc_partials = pl.pallas_call(
    _tc_body,
    grid=(TC_NBLK,),
    in_specs=[
        pl.BlockSpec((1, 1, TC_BLK), lambda i: (TC_OFF + i, 0, 0)),
        pl.BlockSpec((TC_BLK, D), lambda i: (TC_OFF + i, 0)),
        pl.BlockSpec((D, L), lambda i: (0, 0)),
    ],
    out_specs=[
        pl.BlockSpec((SEGS, L), lambda i: (0, 0)),
        pl.BlockSpec((SEGS, L), lambda i: (0, 0)),
    ],
    out_shape=[jax.ShapeDtypeStruct((SEGS, L), jnp.float32)] * 2,
    compiler_params=pltpu.CompilerParams(
        dimension_semantics=("arbitrary",)),
)


@functools.partial(
    pl.kernel,
    out_type=jax.ShapeDtypeStruct((SEGS, L), jnp.float32),
    mesh=_mesh(),
    scratch_types=[
        pltpu.VMEM((NW, 2, L), jnp.float32),
        pltpu.VMEM((NW, 2, L), jnp.float32),
        pltpu.VMEM((2, L), jnp.float32),
        pltpu.VMEM((2, L), jnp.float32),
        pltpu.VMEM((L,), jnp.float32),
        pltpu.VMEM((2 * L,), jnp.float32),
        pltpu.VMEM((2, L), jnp.float32),
    ],
)
def _finish(part, cnts, tcy, tcc, bt, out, pbuf, cbuf, tybuf, tcbuf, bbuf,
            red, obuf):
    w = lax.axis_index("s") * NC + lax.axis_index("c")
    seg0 = w * 2
    pltpu.sync_copy(part.at[:, pl.ds(seg0, 2), :], pbuf)
    pltpu.sync_copy(cnts.at[:, pl.ds(seg0, 2), :], cbuf)
    pltpu.sync_copy(tcy.at[pl.ds(seg0, 2)], tybuf)
    pltpu.sync_copy(tcc.at[pl.ds(seg0, 2)], tcbuf)
    pltpu.sync_copy(bt, bbuf)

    zrow = jnp.zeros((L,), jnp.float32)

    def pbody(p, carry):
        s0, c0, s1, c1 = carry
        return (s0 + pbuf[p, 0], c0 + cbuf[p, 0],
                s1 + pbuf[p, 1], c1 + cbuf[p, 1])

    s0, c0, s1, c1 = lax.fori_loop(0, NW, pbody, (zrow, zrow, zrow, zrow))
    s0, c0 = s0 + tybuf[0], c0 + tcbuf[0]
    s1, c1 = s1 + tybuf[1], c1 + tcbuf[1]

    bias = bbuf[...]
    red[pl.ds(L, L)] = zrow
    for i, (s, c) in enumerate(((s0, c0), (s1, c1))):
        red[pl.ds(0, L)] = s
        for sh in (8, 4, 2, 1):
            red[pl.ds(0, L)] = red[pl.ds(0, L)] + red[pl.ds(sh, L)]
        tot = red[pl.ds(0, L)]
        obuf[i] = tot / jnp.maximum(c, 1.0) + bias

    pltpu.sync_copy(obuf, out.at[pl.ds(seg0, 2)])


def kernel(nodes, edges, receivers, senders, global_latent, node_graph_idx,
           edge_graph_idx, W, b):
    flat_nodes = nodes.reshape(B * N, D)
    flat_idx = (node_graph_idx
                + (jnp.arange(B, dtype=jnp.int32) * G)[:, None]).reshape(-1)
    wt = W.reshape(D)
    we1 = jnp.pad(W.reshape(D, 1), ((0, 0), (0, L - 1)))
    bt = jnp.broadcast_to(b.astype(jnp.float32), (L,))
    idx3 = flat_idx.reshape(TOTAL // TC_BLK, 1, TC_BLK)
    part, cnt = _partials(flat_nodes, flat_idx, wt)
    tcy, tcc = _tc_partials(idx3, flat_nodes, we1)
    res = _finish(part, cnt, tcy, tcc, bt)
    return res.reshape(B, G, L)[..., :1]
